# baseline (speedup 1.0000x reference)
import functools

import jax
import jax.numpy as jnp
from jax import lax
from jax.experimental import pallas as pl
from jax.experimental.pallas import tpu as pltpu
from jax.experimental.pallas import tpu_sc as plsc


def _sc_gather(table, idx3, n_total):
    NW, NCH, CH = idx3.shape
    D = table.shape[1]
    n_per_w = NCH * CH
    mesh = plsc.VectorSubcoreMesh(core_axis_name="c", subcore_axis_name="s")
    info = plsc.get_sparse_core_info()
    nc = info.num_cores

    @functools.partial(
        pl.kernel,
        mesh=mesh,
        out_type=jax.ShapeDtypeStruct((n_total, D), jnp.float32),
        scratch_types=[
            pltpu.VMEM((NCH, CH), jnp.int32),
            pltpu.VMEM((n_per_w, D), jnp.float32),
            pltpu.SemaphoreType.DMA,
        ],
        compiler_params=pltpu.CompilerParams(use_tc_tiling_on_sc=False),
    )
    def k(table_hbm, idx_hbm, out_hbm, idx_v, rows_v, sem):
        wid = lax.axis_index("s") * nc + lax.axis_index("c")
        pltpu.sync_copy(idx_hbm.at[wid], idx_v)
        copies = []
        for j in range(NCH):
            copies.append(
                pltpu.async_copy(
                    table_hbm.at[idx_v.at[j]],
                    rows_v.at[pl.ds(j * CH, CH)],
                    sem,
                )
            )
        for c in copies:
            c.wait()
        pltpu.sync_copy(rows_v, out_hbm.at[pl.ds(wid * n_per_w, n_per_w)])

    return k(table, idx3)


def _attn_body(emb_ref, pos_ref, wk_ref, wq_ref, wv_ref, wl_ref, bl_ref,
               mask_ref, out_ref, *, scale):
    e = emb_ref[...] + pos_ref[...]
    q = jnp.dot(e, wq_ref[...], preferred_element_type=jnp.float32)
    k = jnp.dot(e, wk_ref[...], preferred_element_type=jnp.float32)
    v = jnp.dot(e, wv_ref[...], preferred_element_type=jnp.float32)
    wei = lax.dot_general(q, k, (((1,), (1,)), ((), ())),
                          preferred_element_type=jnp.float32)
    wei = wei * scale + mask_ref[...]
    m = jnp.max(wei, axis=1, keepdims=True)
    p = jnp.exp(wei - m)
    s = jnp.sum(p, axis=1, keepdims=True)
    o = jnp.dot(p, v, preferred_element_type=jnp.float32) / s
    out_ref[...] = jnp.dot(o, wl_ref[...],
                           preferred_element_type=jnp.float32) + bl_ref[...]


def _tc_attn_logits(emb2d, pos_tiled, Wk, Wq, Wv, Wl, bl2d, mask_add, bb, T):
    N, D = emb2d.shape
    V = Wl.shape[1]
    R = bb * T
    grid = N // R
    scale = float(D) ** -0.5
    return pl.pallas_call(
        functools.partial(_attn_body, scale=scale),
        grid=(grid,),
        in_specs=[
            pl.BlockSpec((R, D), lambda i: (i, 0)),
            pl.BlockSpec((R, D), lambda i: (0, 0)),
            pl.BlockSpec((D, D), lambda i: (0, 0)),
            pl.BlockSpec((D, D), lambda i: (0, 0)),
            pl.BlockSpec((D, D), lambda i: (0, 0)),
            pl.BlockSpec((D, V), lambda i: (0, 0)),
            pl.BlockSpec((1, V), lambda i: (0, 0)),
            pl.BlockSpec((R, R), lambda i: (0, 0)),
        ],
        out_specs=pl.BlockSpec((R, V), lambda i: (i, 0)),
        out_shape=jax.ShapeDtypeStruct((N, V), jnp.float32),
        compiler_params=pltpu.CompilerParams(
            dimension_semantics=("parallel",),
        ),
    )(emb2d, pos_tiled, Wk, Wq, Wv, Wl, bl2d, mask_add)


def kernel(x, tok_table, pos_table, Wk, Wq, Wv, Wl, bl):
    B, T = x.shape
    V, D = tok_table.shape
    N = B * T

    BB = 8
    R = BB * T

    info = plsc.get_sparse_core_info()
    NW = info.num_cores * info.num_subcores
    n_per_w = N // NW
    CH = 100
    NCH = n_per_w // CH
    idx3 = x.astype(jnp.int32).reshape(NW, NCH, CH)
    emb2d = _sc_gather(tok_table, idx3, N)

    pos_tiled = jnp.tile(pos_table, (BB, 1))
    r = jnp.arange(R)
    bidx, t = r // T, r % T
    causal = (bidx[:, None] == bidx[None, :]) & (t[:, None] >= t[None, :])
    mask_add = jnp.where(causal, 0.0, -1e30).astype(jnp.float32)
    logits2d = _tc_attn_logits(emb2d, pos_tiled, Wk, Wq, Wv, Wl,
                               bl.reshape(1, V), mask_add, BB, T)
    return logits2d.reshape(B, T, V)
